# baseline (speedup 1.0000x reference)
import functools

import jax
import jax.numpy as jnp
import numpy as np
from jax import lax
from jax.experimental import pallas as pl
from jax.experimental.pallas import tpu as pltpu
from jax.experimental.pallas import tpu_sc as plsc

_POOL = 1024
_LEN = 8
_DIM = 768
_TOPK = 4
_B = 4


_NCHUNK = 4
_PCHUNK = _POOL // _NCHUNK


def _select_body(cls_ref, key_ref, g_ref, out_ref, sim_ref):
    i = pl.program_id(0)
    q = cls_ref[...]
    qn = q * lax.rsqrt(jnp.maximum(jnp.sum(q * q, axis=1, keepdims=True), 1e-12))
    k = key_ref[...]
    kn = k * lax.rsqrt(jnp.maximum(jnp.sum(k * k, axis=1, keepdims=True), 1e-12))
    sim_blk = lax.dot_general(
        qn, kn, (((1,), (1,)), ((), ())),
        preferred_element_type=jnp.float32, precision=lax.Precision.HIGHEST,
    )
    sim_ref[pl.ds(i, 1)] = sim_blk[None]

    @pl.when(i == _NCHUNK - 1)
    def _():
        sims = [sim_ref[ci] for ci in range(_NCHUNK)]
        col = lax.broadcasted_iota(jnp.int32, (_B, _PCHUNK), 1)
        outcol = lax.broadcasted_iota(jnp.int32, (_B, 128), 1)
        big = jnp.int32(2 * _POOL)
        acc = jnp.zeros((_B, 128), jnp.int32)
        for r in range(_TOPK):
            zs = [sims[ci] + g_ref[r * _B:(r + 1) * _B,
                                   ci * _PCHUNK:(ci + 1) * _PCHUNK]
                  for ci in range(_NCHUNK)]
            ms = [jnp.max(z, axis=1, keepdims=True) for z in zs]
            m = ms[0]
            for t in ms[1:]:
                m = jnp.maximum(m, t)
            idx = jnp.full((_B, 1), big, jnp.int32)
            for ci in range(_NCHUNK):
                cand = jnp.min(
                    jnp.where(zs[ci] >= m, col + ci * _PCHUNK, big),
                    axis=1, keepdims=True)
                idx = jnp.minimum(idx, cand)
            acc = acc + jnp.where(outcol == 8 * r, idx, 0)
            sims = [jnp.where(col + ci * _PCHUNK == idx, s - 1000.0, s)
                    for ci, s in enumerate(sims)]
        out_ref[...] = acc


_select = pl.pallas_call(
    _select_body,
    grid=(_NCHUNK,),
    in_specs=[
        pl.BlockSpec((_B, _DIM), lambda i: (0, 0)),
        pl.BlockSpec((_PCHUNK, _DIM), lambda i: (i, 0)),
        pl.BlockSpec((_TOPK * _B, _POOL), lambda i: (0, 0)),
    ],
    out_specs=pl.BlockSpec((_B, 128), lambda i: (0, 0)),
    out_shape=jax.ShapeDtypeStruct((_B, 128), jnp.int32),
    scratch_shapes=[pltpu.VMEM((_NCHUNK, _B, _PCHUNK), jnp.float32)],
)

_ROWS_PER_SC = (_B * _TOPK) // 2


@functools.cache
def _make_gather():
    @functools.partial(
        pl.kernel,
        out_type=jax.ShapeDtypeStruct((_B * _TOPK, _LEN, _DIM), jnp.float32),
        mesh=plsc.VectorSubcoreMesh(core_axis_name="c", subcore_axis_name="s"),
        scratch_types=[
            pltpu.VMEM((1,), jnp.int32),
            pltpu.VMEM((1, _LEN, _DIM), jnp.float32),
            pltpu.SemaphoreType.DMA,
        ],
    )
    def _gather(idxm_hbm, table_hbm, out_hbm, idx_v, rows_v, sem):
        c = lax.axis_index("c")
        s = lax.axis_index("s")
        wid = s * 2 + c

        @pl.when(wid < _B * _TOPK)
        def _():
            b = wid // _TOPK
            r = wid % _TOPK
            pltpu.sync_copy(idxm_hbm.at[b, pl.ds(8 * r, 1)], idx_v)
            pltpu.async_copy(table_hbm.at[idx_v], rows_v, sem).wait()
            pltpu.sync_copy(rows_v, out_hbm.at[pl.ds(wid, 1)])

    return _gather


def _gumbel_const():
    with jax.default_device(jax.devices("cpu")[0]):
        gkey = jax.random.key(42)
        gs = []
        for _ in range(_TOPK):
            gkey, sub = jax.random.split(gkey)
            u = jax.random.uniform(sub, (_B, _POOL), minval=1e-20, maxval=1.0)
            gs.append(-jnp.log(-jnp.log(u) + 1e-20))
        return np.concatenate([np.asarray(x) for x in gs], axis=0)


_G_NOISE = _gumbel_const()


def kernel(x_embed, cls_features, prompt, prompt_key):
    g = jnp.asarray(_G_NOISE)
    idx_mat = _select(cls_features, prompt_key, g)
    rows = _make_gather()(idx_mat, prompt)
    return rows.reshape(_B, _TOPK * _LEN, _DIM)


# speedup vs baseline: 1.0135x; 1.0135x over previous
import functools

import jax
import jax.numpy as jnp
import numpy as np
from jax import lax
from jax.experimental import pallas as pl
from jax.experimental.pallas import tpu as pltpu
from jax.experimental.pallas import tpu_sc as plsc

_POOL = 1024
_LEN = 8
_DIM = 768
_TOPK = 4
_B = 4


def _select_body(cls_ref, key_ref, g_ref, out_ref):
    q = cls_ref[...]
    k = key_ref[...]
    qn = q * lax.rsqrt(jnp.maximum(jnp.sum(q * q, axis=1, keepdims=True), 1e-12))
    kn = k * lax.rsqrt(jnp.maximum(jnp.sum(k * k, axis=1, keepdims=True), 1e-12))
    sim = lax.dot_general(
        qn, kn, (((1,), (1,)), ((), ())),
        preferred_element_type=jnp.float32, precision=lax.Precision.HIGHEST,
    )
    col = lax.broadcasted_iota(jnp.int32, (_B, _POOL), 1)
    outcol = lax.broadcasted_iota(jnp.int32, (_B, 128), 1)
    acc = jnp.zeros((_B, 128), jnp.int32)
    for r in range(_TOPK):
        z = sim + g_ref[r * _B:(r + 1) * _B, :]
        m = jnp.max(z, axis=1, keepdims=True)
        idx = jnp.min(jnp.where(z >= m, col, _POOL), axis=1, keepdims=True)
        acc = acc + jnp.where(outcol == 8 * r, idx, 0)
        sim = jnp.where(col == idx, sim - 1000.0, sim)
    out_ref[...] = acc


_select = pl.pallas_call(
    _select_body,
    out_shape=jax.ShapeDtypeStruct((_B, 128), jnp.int32),
)

_ROWS_PER_SC = (_B * _TOPK) // 2


@functools.cache
def _make_gather():
    @functools.partial(
        pl.kernel,
        out_type=jax.ShapeDtypeStruct((_B * _TOPK, _LEN, _DIM), jnp.float32),
        mesh=plsc.VectorSubcoreMesh(core_axis_name="c", subcore_axis_name="s"),
        scratch_types=[
            pltpu.VMEM((1,), jnp.int32),
            pltpu.VMEM((1, _LEN, _DIM), jnp.float32),
            pltpu.SemaphoreType.DMA,
        ],
    )
    def _gather(idxm_hbm, table_hbm, out_hbm, idx_v, rows_v, sem):
        c = lax.axis_index("c")
        s = lax.axis_index("s")
        wid = s * 2 + c

        @pl.when(wid < _B * _TOPK)
        def _():
            b = wid // _TOPK
            r = wid % _TOPK
            pltpu.sync_copy(idxm_hbm.at[b, pl.ds(8 * r, 1)], idx_v)
            pltpu.async_copy(table_hbm.at[idx_v], rows_v, sem).wait()
            pltpu.sync_copy(rows_v, out_hbm.at[pl.ds(wid, 1)])

    return _gather


def _gumbel_const():
    with jax.default_device(jax.devices("cpu")[0]):
        gkey = jax.random.key(42)
        gs = []
        for _ in range(_TOPK):
            gkey, sub = jax.random.split(gkey)
            u = jax.random.uniform(sub, (_B, _POOL), minval=1e-20, maxval=1.0)
            gs.append(-jnp.log(-jnp.log(u) + 1e-20))
        return np.concatenate([np.asarray(x) for x in gs], axis=0)


_G_NOISE = _gumbel_const()


def kernel(x_embed, cls_features, prompt, prompt_key):
    g = jnp.asarray(_G_NOISE)
    idx_mat = _select(cls_features, prompt_key, g)
    rows = _make_gather()(idx_mat, prompt)
    return rows.reshape(_B, _TOPK * _LEN, _DIM)


# speedup vs baseline: 1.0171x; 1.0036x over previous
import functools

import jax
import jax.numpy as jnp
import numpy as np
from jax import lax
from jax.experimental import pallas as pl
from jax.experimental.pallas import tpu as pltpu
from jax.experimental.pallas import tpu_sc as plsc

_POOL = 1024
_LEN = 8
_DIM = 768
_TOPK = 4
_B = 4


def _select_body(cls_ref, key_ref, g_ref, out_ref):
    q = cls_ref[...]
    k = key_ref[...]
    qn = q * lax.rsqrt(jnp.maximum(jnp.sum(q * q, axis=1, keepdims=True), 1e-12))
    kn = k * lax.rsqrt(jnp.maximum(jnp.sum(k * k, axis=1, keepdims=True), 1e-12))
    sim = lax.dot_general(
        qn, kn, (((1,), (1,)), ((), ())),
        preferred_element_type=jnp.float32, precision=lax.Precision.HIGHEST,
    )
    col = lax.broadcasted_iota(jnp.int32, (_B, _POOL), 1)
    outcol = lax.broadcasted_iota(jnp.int32, (_B, 128), 1)
    acc = jnp.zeros((_B, 128), jnp.int32)
    for r in range(_TOPK):
        z = sim + g_ref[r * _B:(r + 1) * _B, :]
        m = jnp.max(z, axis=1, keepdims=True)
        idx = jnp.min(jnp.where(z >= m, col, _POOL), axis=1, keepdims=True)
        acc = acc + jnp.where(outcol == 8 * r, idx, 0)
        sim = jnp.where(col == idx, sim - 1000.0, sim)
    out_ref[...] = acc


_select = pl.pallas_call(
    _select_body,
    out_shape=jax.ShapeDtypeStruct((_B, 128), jnp.int32),
)

_ROWS_PER_SC = (_B * _TOPK) // 2


@functools.cache
def _make_gather():
    @functools.partial(
        pl.kernel,
        out_type=jax.ShapeDtypeStruct((_B * _TOPK, _LEN, _DIM), jnp.float32),
        mesh=plsc.VectorSubcoreMesh(core_axis_name="c", subcore_axis_name="s"),
        scratch_types=[
            pltpu.VMEM((1,), jnp.int32),
            pltpu.VMEM((1, _LEN, _DIM), jnp.float32),
            pltpu.SemaphoreType.DMA,
        ],
    )
    def _gather(idxm_hbm, table_hbm, out_hbm, idx_v, rows_v, sem):
        c = lax.axis_index("c")
        s = lax.axis_index("s")
        wid = s * 2 + c

        @pl.when(wid < _B * _TOPK)
        def _():
            b = wid // _TOPK
            r = wid % _TOPK
            pltpu.sync_copy(idxm_hbm.at[b, pl.ds(8 * r, 1)], idx_v)
            pltpu.async_copy(table_hbm.at[idx_v], rows_v, sem).wait()
            pltpu.sync_copy(rows_v, out_hbm.at[pl.ds(wid, 1)])

    return _gather


def _rotl32(x, d):
    return ((x << np.uint32(d)) | (x >> np.uint32(32 - d))).astype(np.uint32)


def _threefry2x32(k1, k2, x0, x1):
    x0 = np.asarray(x0, np.uint32).copy()
    x1 = np.asarray(x1, np.uint32).copy()
    ks0 = np.uint32(k1)
    ks1 = np.uint32(k2)
    ks2 = np.uint32(ks0 ^ ks1 ^ np.uint32(0x1BD11BDA))
    rots1 = (13, 15, 26, 6)
    rots2 = (17, 29, 16, 24)
    x0 += ks0
    x1 += ks1
    inject = [(ks1, ks2), (ks2, ks0), (ks0, ks1), (ks1, ks2), (ks2, ks0)]
    for g in range(5):
        for r in rots1 if g % 2 == 0 else rots2:
            x0 += x1
            x1 = _rotl32(x1, r)
            x1 ^= x0
        a, b = inject[g]
        x0 += a
        x1 += b + np.uint32(g + 1)
    return x0, x1


def _gumbel_const():
    key = (np.uint32(0), np.uint32(42))
    gs = []
    for _ in range(_TOPK):
        b1, b2 = _threefry2x32(key[0], key[1],
                               np.zeros(2, np.uint32),
                               np.arange(2, dtype=np.uint32))
        key, sub = (b1[0], b2[0]), (b1[1], b2[1])
        n = _B * _POOL
        b1, b2 = _threefry2x32(sub[0], sub[1],
                               np.zeros(n, np.uint32),
                               np.arange(n, dtype=np.uint32))
        bits = b1 ^ b2
        fb = (bits >> np.uint32(9)) | np.uint32(0x3F800000)
        f = fb.view(np.float32) - np.float32(1.0)
        u = f * (np.float32(1.0) - np.float32(1e-20)) + np.float32(1e-20)
        u = np.maximum(np.float32(1e-20), u).reshape(_B, _POOL)
        l1 = np.log(u, dtype=np.float32)
        gs.append(-np.log((-l1 + np.float32(1e-20)).astype(np.float32),
                          dtype=np.float32))
    return np.concatenate(gs, axis=0).astype(np.float32)


_G_NOISE = _gumbel_const()


def kernel(x_embed, cls_features, prompt, prompt_key):
    g = jnp.asarray(_G_NOISE)
    idx_mat = _select(cls_features, prompt_key, g)
    rows = _make_gather()(idx_mat, prompt)
    return rows.reshape(_B, _TOPK * _LEN, _DIM)
